# baseline (speedup 1.0000x reference)
import functools

import jax
import jax.numpy as jnp
from jax import lax
from jax.experimental import pallas as pl
from jax.experimental.pallas import tpu as pltpu
from jax.experimental.pallas import tpu_sc as plsc

B, T, V, E, H = 128, 32, 10000, 512, 512
G4 = 4 * H

NC, NS = 2, 16
NW = NC * NS
ROWS = B * T

TA = 8
TB = T - TA


def _make_gather_body(rpw):
    def _gather_body(emb_hbm, idx_hbm, out_hbm, idx_v, rows_v, sem):
        wid = lax.axis_index("s") * NC + lax.axis_index("c")
        base = wid * rpw
        pltpu.sync_copy(idx_hbm.at[pl.ds(base, rpw)], idx_v)
        pltpu.async_copy(emb_hbm.at[idx_v], rows_v, sem).wait()
        pltpu.sync_copy(rows_v, out_hbm.at[pl.ds(base, rpw)])
    return _gather_body


def _sc_gather(emb, idx, n_rows):
    rpw = n_rows // NW
    mesh = plsc.VectorSubcoreMesh(
        core_axis_name="c", subcore_axis_name="s",
        num_cores=NC, num_subcores=NS)
    return pl.kernel(
        _make_gather_body(rpw),
        out_type=jax.ShapeDtypeStruct((n_rows, E), jnp.float32),
        mesh=mesh,
        scratch_types=[
            pltpu.VMEM((rpw,), jnp.int32),
            pltpu.VMEM((rpw, E), jnp.float32),
            pltpu.SemaphoreType.DMA,
        ],
    )(emb, idx)


BV = 2048
NVBLK = 5


def _lstm_step(x_blk, whi_ref, b_ref, xh_scr, c_scr):
    xh = jnp.concatenate([x_blk.astype(jnp.bfloat16), xh_scr[...]], axis=1)
    g = (jnp.dot(xh, whi_ref[...], preferred_element_type=jnp.float32)
         + b_ref[...])
    gi = jax.nn.sigmoid(g[:, 0:H])
    gf = jax.nn.sigmoid(g[:, H:2 * H])
    gg = jnp.tanh(g[:, 2 * H:3 * H])
    go = jax.nn.sigmoid(g[:, 3 * H:4 * H])
    c = gf * c_scr[...] + gi * gg
    h = go * jnp.tanh(c)
    c_scr[...] = c
    xh_scr[...] = h.astype(jnp.bfloat16)
    return h


def _phase_a_body(x_ref, whi_ref, b_ref, hb_out, c_out, xh_scr, c_scr):
    t = pl.program_id(0)

    @pl.when(t == 0)
    def _():
        xh_scr[...] = jnp.zeros((B, H), jnp.bfloat16)
        c_scr[...] = jnp.zeros_like(c_scr)

    _lstm_step(x_ref[0], whi_ref, b_ref, xh_scr, c_scr)

    @pl.when(t == TA - 1)
    def _():
        hb_out[...] = xh_scr[...]
        c_out[...] = c_scr[...]


def _phase_a(x1, Whi, b):
    return pl.pallas_call(
        _phase_a_body,
        grid=(TA,),
        in_specs=[
            pl.BlockSpec((1, B, E), lambda t: (t, 0, 0)),
            pl.BlockSpec((E + H, G4), lambda t: (0, 0)),
            pl.BlockSpec((1, G4), lambda t: (0, 0)),
        ],
        out_specs=[
            pl.BlockSpec((B, H), lambda t: (0, 0)),
            pl.BlockSpec((B, H), lambda t: (0, 0)),
        ],
        out_shape=[
            jax.ShapeDtypeStruct((B, H), jnp.bfloat16),
            jax.ShapeDtypeStruct((B, H), jnp.float32),
        ],
        scratch_shapes=[
            pltpu.VMEM((B, H), jnp.bfloat16),
            pltpu.VMEM((B, H), jnp.float32),
        ],
    )(x1, Whi, b)


def _phase_b_body(x_ref, whi_ref, b_ref, bout_ref, hb_ref, c_ref, wout_hbm,
                  out_ref, xh_scr, c_scr, wout_vmem, dma_sem):
    t = pl.program_id(0)

    @pl.when(t == 0)
    def _():
        xh_scr[...] = hb_ref[...]
        c_scr[...] = c_ref[...]
        pltpu.make_async_copy(wout_hbm, wout_vmem, dma_sem).start()

    h = _lstm_step(x_ref[0], whi_ref, b_ref, xh_scr, c_scr)

    @pl.when(t == TB - 1)
    def _():
        pltpu.make_async_copy(wout_hbm, wout_vmem, dma_sem).wait()
        hb = h.astype(jnp.bfloat16)
        for j in range(NVBLK):
            lo = j * BV
            w = min(BV, V - lo)
            wblk = wout_vmem[lo:lo + w, :].astype(jnp.bfloat16)
            acc = jax.lax.dot_general(
                wblk, hb, (((1,), (1,)), ((), ())),
                preferred_element_type=jnp.float32)
            col = bout_ref[0, lo:lo + w].reshape(w, 1)
            out_ref[lo:lo + w, :] = acc + col


def _phase_b(x2, Whi, b, b_out, hb, c, W_out_t):
    return pl.pallas_call(
        _phase_b_body,
        grid=(TB,),
        in_specs=[
            pl.BlockSpec((1, B, E), lambda t: (t, 0, 0)),
            pl.BlockSpec((E + H, G4), lambda t: (0, 0)),
            pl.BlockSpec((1, G4), lambda t: (0, 0)),
            pl.BlockSpec((1, V), lambda t: (0, 0)),
            pl.BlockSpec((B, H), lambda t: (0, 0)),
            pl.BlockSpec((B, H), lambda t: (0, 0)),
            pl.BlockSpec(memory_space=pltpu.HBM),
        ],
        out_specs=pl.BlockSpec((V, B), lambda t: (0, 0)),
        out_shape=jax.ShapeDtypeStruct((V, B), jnp.float32),
        scratch_shapes=[
            pltpu.VMEM((B, H), jnp.bfloat16),
            pltpu.VMEM((B, H), jnp.float32),
            pltpu.VMEM((V, H), jnp.float32),
            pltpu.SemaphoreType.DMA,
        ],
    )(x2, Whi, b, b_out, hb, c, W_out_t)


def kernel(surf, emb, Wi, Wh, b, W_root, b_root, cb_root, W_ord, b_ord,
           cb_ord, W_out, b_out):
    idx = jnp.transpose(surf).reshape(ROWS).astype(jnp.int32)
    n1 = TA * B
    x1 = _sc_gather(emb, idx[:n1], n1).reshape(TA, B, E)
    x2 = _sc_gather(emb, idx[n1:], ROWS - n1).reshape(TB, B, E)
    whi = jnp.concatenate([Wi, Wh], axis=0).astype(jnp.bfloat16)
    b2 = b.reshape(1, G4)
    hb, c = _phase_a(x1, whi, b2)
    logits_t = _phase_b(x2, whi, b2, b_out.reshape(1, V), hb, c,
                        jnp.transpose(W_out))
    return jnp.transpose(logits_t).reshape(B, 1, V)


# speedup vs baseline: 1.0115x; 1.0115x over previous
import functools

import jax
import jax.numpy as jnp
from jax import lax
from jax.experimental import pallas as pl
from jax.experimental.pallas import tpu as pltpu
from jax.experimental.pallas import tpu_sc as plsc

B, T, V, E, H = 128, 32, 10000, 512, 512
G4 = 4 * H

NC, NS = 2, 16
NW = NC * NS
ROWS = B * T
RPW = ROWS // NW


def _gather_body(emb_hbm, idx_hbm, out_hbm, idx_v, rows_v, sem):
    wid = lax.axis_index("s") * NC + lax.axis_index("c")
    base = wid * RPW
    pltpu.sync_copy(idx_hbm.at[pl.ds(base, RPW)], idx_v)
    pltpu.async_copy(emb_hbm.at[idx_v], rows_v, sem).wait()
    pltpu.sync_copy(rows_v, out_hbm.at[pl.ds(base, RPW)])


def _sc_gather(emb, idx):
    mesh = plsc.VectorSubcoreMesh(
        core_axis_name="c", subcore_axis_name="s",
        num_cores=NC, num_subcores=NS)
    return pl.kernel(
        _gather_body,
        out_type=jax.ShapeDtypeStruct((ROWS, E), jnp.float32),
        mesh=mesh,
        scratch_types=[
            pltpu.VMEM((RPW,), jnp.int32),
            pltpu.VMEM((RPW, E), jnp.float32),
            pltpu.SemaphoreType.DMA,
        ],
    )(emb, idx)


BV = 2048
NVBLK = 5


def _fused_body(x_ref, wi_ref, wh_ref, b_ref, bout_ref, wout_hbm,
                out_ref, hb_scr, c_scr, wout_vmem, dma_sem):
    t = pl.program_id(0)

    @pl.when(t == 0)
    def _():
        hb_scr[...] = jnp.zeros((B, H), jnp.bfloat16)
        c_scr[...] = jnp.zeros_like(c_scr)
        pltpu.make_async_copy(wout_hbm, wout_vmem, dma_sem).start()

    x = x_ref[0].astype(jnp.bfloat16)
    g = (jnp.dot(x, wi_ref[...], preferred_element_type=jnp.float32)
         + jnp.dot(hb_scr[...], wh_ref[...],
                   preferred_element_type=jnp.float32)
         + b_ref[...])
    gi = jax.nn.sigmoid(g[:, 0:H])
    gf = jax.nn.sigmoid(g[:, H:2 * H])
    gg = jnp.tanh(g[:, 2 * H:3 * H])
    go = jax.nn.sigmoid(g[:, 3 * H:4 * H])
    c = gf * c_scr[...] + gi * gg
    h = go * jnp.tanh(c)
    c_scr[...] = c
    hb_scr[...] = h.astype(jnp.bfloat16)

    @pl.when(t == T - 1)
    def _():
        pltpu.make_async_copy(wout_hbm, wout_vmem, dma_sem).wait()
        hb = h.astype(jnp.bfloat16)
        for j in range(NVBLK):
            lo = j * BV
            w = min(BV, V - lo)
            wblk = wout_vmem[lo:lo + w, :].astype(jnp.bfloat16)
            acc = jax.lax.dot_general(
                wblk, hb, (((1,), (1,)), ((), ())),
                preferred_element_type=jnp.float32)
            col = bout_ref[0, lo:lo + w].reshape(w, 1)
            out_ref[lo:lo + w, :] = acc + col


def _fused(x, Wi_bf, Wh_bf, b, W_out_t, b_out):
    return pl.pallas_call(
        _fused_body,
        grid=(T,),
        in_specs=[
            pl.BlockSpec((1, B, E), lambda t: (t, 0, 0)),
            pl.BlockSpec((E, G4), lambda t: (0, 0)),
            pl.BlockSpec((H, G4), lambda t: (0, 0)),
            pl.BlockSpec((1, G4), lambda t: (0, 0)),
            pl.BlockSpec((1, V), lambda t: (0, 0)),
            pl.BlockSpec(memory_space=pltpu.HBM),
        ],
        out_specs=pl.BlockSpec((V, B), lambda t: (0, 0)),
        out_shape=jax.ShapeDtypeStruct((V, B), jnp.float32),
        scratch_shapes=[
            pltpu.VMEM((B, H), jnp.bfloat16),
            pltpu.VMEM((B, H), jnp.float32),
            pltpu.VMEM((V, H), jnp.float32),
            pltpu.SemaphoreType.DMA,
        ],
    )(x, Wi_bf, Wh_bf, b, b_out, W_out_t)


def kernel(surf, emb, Wi, Wh, b, W_root, b_root, cb_root, W_ord, b_ord,
           cb_ord, W_out, b_out):
    idx = jnp.transpose(surf).reshape(ROWS).astype(jnp.int32)
    x = _sc_gather(emb, idx).reshape(T, B, E)
    logits_t = _fused(x, Wi.astype(jnp.bfloat16), Wh.astype(jnp.bfloat16),
                      b.reshape(1, G4), jnp.transpose(W_out),
                      b_out.reshape(1, V))
    return jnp.transpose(logits_t).reshape(B, 1, V)
